# Initial kernel scaffold; baseline (speedup 1.0000x reference)
import jax
import jax.numpy as jnp
from jax.experimental import pallas as pl


def kernel(x_bs, tok_weight, pos_weight):
    raise NotImplementedError("write your pallas kernel here")


# speedup vs baseline: 1.3202x; 1.3202x over previous
import functools
import math

import jax
import jax.numpy as jnp
import numpy as np
from jax import lax
from jax.experimental import pallas as pl
from jax.experimental.pallas import tpu as pltpu
from jax.experimental.pallas import tpu_sc as plsc

VOCAB = 100000
SEQ_LEN = 2048
EMB = 128
BATCH = 4

NC = 2
NS = 16
NW = NC * NS
ROWS = BATCH * SEQ_LEN
BPW = ROWS // NW
CH = 128
NCH = BPW // CH
LANES = 16
SCALE = np.float32(math.sqrt(EMB))

_mesh = plsc.VectorSubcoreMesh(core_axis_name="c", subcore_axis_name="s")


@functools.partial(
    pl.kernel,
    out_type=jax.ShapeDtypeStruct((ROWS, EMB), jnp.float32),
    mesh=_mesh,
    scratch_types=[
        pltpu.VMEM((NCH, CH), jnp.int32),
        pltpu.VMEM((BPW, EMB), jnp.float32),
        pltpu.VMEM((BPW, EMB), jnp.float32),
        pltpu.SemaphoreType.DMA,
    ],
)
def _emb_kernel(idx_hbm, tok_hbm, pos_hbm, out_hbm, idx_v, rows_v, pos_v, sem):
    wid = lax.axis_index("s") * NC + lax.axis_index("c")
    base = wid * BPW
    s_base = lax.rem(base, SEQ_LEN)

    pltpu.sync_copy(idx_hbm.at[pl.ds(wid * NCH, NCH)], idx_v)

    copies = []
    for j in range(NCH):
        copies.append(
            pltpu.async_copy(
                tok_hbm.at[idx_v.at[j]], rows_v.at[pl.ds(j * CH, CH)], sem
            )
        )
    pltpu.sync_copy(pos_hbm.at[pl.ds(s_base, BPW)], pos_v)
    for c in copies:
        c.wait()

    def body(i, carry):
        for j in range(EMB // LANES):
            sl = pl.ds(j * LANES, LANES)
            rows_v[i, sl] = (rows_v[i, sl] + pos_v[i, sl]) * SCALE
        return carry

    lax.fori_loop(0, BPW, body, 0)

    pltpu.sync_copy(rows_v, out_hbm.at[pl.ds(base, BPW)])


def kernel(x_bs, tok_weight, pos_weight):
    idx = x_bs.reshape(NW * NCH, CH)
    out = _emb_kernel(idx, tok_weight, pos_weight)
    return out.reshape(BATCH, SEQ_LEN, EMB)
